# Initial kernel scaffold; baseline (speedup 1.0000x reference)
import jax
import jax.numpy as jnp
from jax.experimental import pallas as pl


def kernel(inputs, edge_index, W1, b1, W2, b2):
    raise NotImplementedError("write your pallas kernel here")


# speedup vs baseline: 7.8438x; 7.8438x over previous
import functools

import jax
import jax.numpy as jnp
from jax import lax
from jax.experimental import pallas as pl
from jax.experimental.pallas import tpu as pltpu
from jax.experimental.pallas import tpu_sc as plsc

N = 10000
E = 160000
F_IN = 256
HID = 512
F_OUT = 256
FH = 128

NT = 16
EPT = 10240
EPAD = EPT * NT
CH = 128
NCHUNK = EPT // CH
RPT_A = 632
RPT_B = N - 15 * RPT_A
NPAD = N + 8

_sc_mesh = plsc.VectorSubcoreMesh(core_axis_name="c", subcore_axis_name="s")


@functools.partial(
    pl.kernel,
    mesh=_sc_mesh,
    out_type=jax.ShapeDtypeStruct((2, N, FH), jnp.float32),
    scratch_types=[
        pltpu.VMEM((CH,), jnp.int32),
        pltpu.VMEM((CH, FH), jnp.float32),
        pltpu.VMEM_SHARED((NPAD, FH), jnp.float32),
    ],
)
def _deg_sc(dst_hbm, zeros_hbm, ones_hbm, degp_hbm, dstb, onesb, accd):
    c = lax.axis_index("c")
    s = lax.axis_index("s")
    r0 = s * RPT_A

    @pl.when(s < NT - 1)
    def _():
        pltpu.sync_copy(zeros_hbm.at[pl.ds(r0, RPT_A)], accd.at[pl.ds(r0, RPT_A)])

    @pl.when(s == NT - 1)
    def _():
        pltpu.sync_copy(zeros_hbm.at[pl.ds(r0, RPT_B)], accd.at[pl.ds(r0, RPT_B)])

    pltpu.sync_copy(ones_hbm, onesb)
    plsc.subcore_barrier()

    def chunk(k, carry):
        e0 = s * EPT + c * (EPT // 2) + k * CH
        pltpu.sync_copy(dst_hbm.at[pl.ds(e0, CH)], dstb)
        pltpu.sync_copy(onesb, accd.at[dstb], add=True)
        return carry

    lax.fori_loop(0, NCHUNK // 2, chunk, 0)
    plsc.subcore_barrier()

    @pl.when(s < NT - 1)
    def _():
        pltpu.sync_copy(accd.at[pl.ds(r0, RPT_A)], degp_hbm.at[c, pl.ds(r0, RPT_A)])

    @pl.when(s == NT - 1)
    def _():
        pltpu.sync_copy(accd.at[pl.ds(r0, RPT_B)], degp_hbm.at[c, pl.ds(r0, RPT_B)])


@functools.partial(
    pl.kernel,
    mesh=_sc_mesh,
    out_type=jax.ShapeDtypeStruct((2 * N, FH), jnp.float32),
    scratch_types=[
        pltpu.VMEM((CH,), jnp.int32),
        pltpu.VMEM((CH,), jnp.int32),
        pltpu.VMEM((CH,), jnp.int32),
        pltpu.VMEM((CH, FH), jnp.float32),
        pltpu.VMEM_SHARED((NPAD, FH), jnp.float32),
        pltpu.SemaphoreType.DMA,
    ],
)
def _agg_sc(xs_hbm, src_hbm, dst_hbm, out_hbm, idxb, adjb, dstb, gbuf, acc, sem):
    c = lax.axis_index("c")
    s = lax.axis_index("s")
    off = c * N
    r0 = s * RPT_A

    @pl.when(s < NT - 1)
    def _():
        pltpu.sync_copy(xs_hbm.at[pl.ds(off + r0, RPT_A)], acc.at[pl.ds(r0, RPT_A)])

    @pl.when(s == NT - 1)
    def _():
        pltpu.sync_copy(xs_hbm.at[pl.ds(off + r0, RPT_B)], acc.at[pl.ds(r0, RPT_B)])

    plsc.subcore_barrier()

    def chunk(k, carry):
        e0 = s * EPT + k * CH
        pltpu.sync_copy(src_hbm.at[pl.ds(e0, CH)], idxb)
        pltpu.sync_copy(dst_hbm.at[pl.ds(e0, CH)], dstb)
        for j in range(CH // 16):
            adjb[pl.ds(j * 16, 16)] = idxb[pl.ds(j * 16, 16)] + off
        pltpu.async_copy(xs_hbm.at[adjb], gbuf, sem).wait()
        pltpu.sync_copy(gbuf, acc.at[dstb], add=True)
        return carry

    lax.fori_loop(0, NCHUNK, chunk, 0)
    plsc.subcore_barrier()

    @pl.when(s < NT - 1)
    def _():
        pltpu.sync_copy(acc.at[pl.ds(r0, RPT_A)], out_hbm.at[pl.ds(off + r0, RPT_A)])

    @pl.when(s == NT - 1)
    def _():
        pltpu.sync_copy(acc.at[pl.ds(r0, RPT_B)], out_hbm.at[pl.ds(off + r0, RPT_B)])


def _scale_body(x_ref, degp_ref, xs_ref, normc_ref):
    d = degp_ref[0, :, 0:1] + degp_ref[1, :, 0:1]
    norm = lax.rsqrt(1.0 + d)
    normc_ref[...] = jnp.broadcast_to(norm, normc_ref.shape)
    xs = x_ref[...] * norm
    xs_ref[0] = xs[:, :FH]
    xs_ref[1] = xs[:, FH:]


def _mm_body(agg_ref, normc_ref, w1_ref, b1_ref, w2_ref, ps_ref):
    nc = normc_ref[...]
    a = jnp.concatenate([agg_ref[0] * nc, agg_ref[1] * nc], axis=1)
    h = jnp.dot(a, w1_ref[...], preferred_element_type=jnp.float32)
    h = jnp.maximum(h + b1_ref[...], 0.0)
    p = jnp.dot(h, w2_ref[...], preferred_element_type=jnp.float32)
    ps_ref[0] = p[:, :FH] * nc
    ps_ref[1] = p[:, FH:] * nc


def _final_body(agg_ref, normc_ref, b2_ref, out_ref):
    nc = normc_ref[...]
    out_ref[...] = jnp.concatenate(
        [agg_ref[0] * nc, agg_ref[1] * nc], axis=1) + b2_ref[...]


_BN = 1000
_G = N // _BN

_scale_call = pl.pallas_call(
    _scale_body,
    grid=(_G,),
    in_specs=[
        pl.BlockSpec((_BN, F_IN), lambda i: (i, 0)),
        pl.BlockSpec((2, _BN, FH), lambda i: (0, i, 0)),
    ],
    out_specs=[
        pl.BlockSpec((2, _BN, FH), lambda i: (0, i, 0)),
        pl.BlockSpec((_BN, FH), lambda i: (i, 0)),
    ],
    out_shape=[
        jax.ShapeDtypeStruct((2, N, FH), jnp.float32),
        jax.ShapeDtypeStruct((N, FH), jnp.float32),
    ],
)

_mm_call = pl.pallas_call(
    _mm_body,
    grid=(_G,),
    in_specs=[
        pl.BlockSpec((2, _BN, FH), lambda i: (0, i, 0)),
        pl.BlockSpec((_BN, FH), lambda i: (i, 0)),
        pl.BlockSpec((F_IN, HID), lambda i: (0, 0)),
        pl.BlockSpec((1, HID), lambda i: (0, 0)),
        pl.BlockSpec((HID, F_OUT), lambda i: (0, 0)),
    ],
    out_specs=pl.BlockSpec((2, _BN, FH), lambda i: (0, i, 0)),
    out_shape=jax.ShapeDtypeStruct((2, N, FH), jnp.float32),
)

_final_call = pl.pallas_call(
    _final_body,
    grid=(_G,),
    in_specs=[
        pl.BlockSpec((2, _BN, FH), lambda i: (0, i, 0)),
        pl.BlockSpec((_BN, FH), lambda i: (i, 0)),
        pl.BlockSpec((1, F_OUT), lambda i: (0, 0)),
    ],
    out_specs=pl.BlockSpec((_BN, F_OUT), lambda i: (i, 0)),
    out_shape=jax.ShapeDtypeStruct((N, F_OUT), jnp.float32),
)


def kernel(inputs, edge_index, W1, b1, W2, b2):
    src = edge_index[0].astype(jnp.int32)
    dst = edge_index[1].astype(jnp.int32)

    npad = EPT - E // NT
    pad_src = (jnp.arange(npad * NT, dtype=jnp.int32) * 401) % N
    pad_dst = N + (jnp.arange(npad * NT, dtype=jnp.int32) % 8)
    src_p = jnp.concatenate(
        [src.reshape(NT, E // NT), pad_src.reshape(NT, npad)], axis=1).reshape(-1)
    dst_p = jnp.concatenate(
        [dst.reshape(NT, E // NT), pad_dst.reshape(NT, npad)], axis=1).reshape(-1)

    zerosw = jnp.zeros((N, FH), jnp.float32)
    onesw = jnp.ones((CH, FH), jnp.float32)

    degp = _deg_sc(dst_p, zerosw, onesw)
    xs2, normc = _scale_call(inputs, degp)
    agg1 = _agg_sc(xs2.reshape(2 * N, FH), src_p, dst_p)
    ps2 = _mm_call(agg1.reshape(2, N, FH), normc, W1,
                   b1.reshape(1, HID), W2)
    agg2 = _agg_sc(ps2.reshape(2 * N, FH), src_p, dst_p)
    out = _final_call(agg2.reshape(2, N, FH), normc,
                      b2.reshape(1, F_OUT))
    return out


# speedup vs baseline: 13.8975x; 1.7718x over previous
import functools

import jax
import jax.numpy as jnp
from jax import lax
from jax.experimental import pallas as pl
from jax.experimental.pallas import tpu as pltpu
from jax.experimental.pallas import tpu_sc as plsc

N = 10000
E = 160000
F_IN = 256
HID = 512
F_OUT = 256
FH = 128

NT = 16
EPT = 10240
EPAD = EPT * NT
CH = 80
NCHUNK = EPT // CH
RPT_A = 632
RPT_B = N - 15 * RPT_A
NPAD = N + 8

_sc_mesh = plsc.VectorSubcoreMesh(core_axis_name="c", subcore_axis_name="s")


@functools.partial(
    pl.kernel,
    mesh=_sc_mesh,
    out_type=jax.ShapeDtypeStruct((2, N, FH), jnp.float32),
    scratch_types=[
        pltpu.VMEM((NCHUNK // 2, CH), jnp.int32),
        pltpu.VMEM((CH, FH), jnp.float32),
        pltpu.VMEM_SHARED((NPAD, FH), jnp.float32),
        pltpu.SemaphoreType.DMA,
    ],
)
def _deg_sc(dst3_hbm, zeros_hbm, ones_hbm, degp_hbm, dstd, onesb, accd, sem):
    c = lax.axis_index("c")
    s = lax.axis_index("s")
    r0 = s * RPT_A

    @pl.when(s < NT - 1)
    def _():
        pltpu.sync_copy(zeros_hbm.at[pl.ds(r0, RPT_A)], accd.at[pl.ds(r0, RPT_A)])

    @pl.when(s == NT - 1)
    def _():
        pltpu.sync_copy(zeros_hbm.at[pl.ds(r0, RPT_B)], accd.at[pl.ds(r0, RPT_B)])

    pltpu.sync_copy(dst3_hbm.at[s, pl.ds(c * (NCHUNK // 2), NCHUNK // 2)], dstd)
    pltpu.sync_copy(ones_hbm, onesb)
    plsc.subcore_barrier()

    def fire(k, carry):
        pltpu.async_copy(onesb, accd.at[dstd.at[k]], sem, add=True)
        return carry

    lax.fori_loop(0, NCHUNK // 2, fire, 0)

    def drain(k, carry):
        pltpu.make_async_copy(onesb, accd.at[dstd.at[0]], sem).wait()
        return carry

    lax.fori_loop(0, NCHUNK // 2, drain, 0)
    plsc.subcore_barrier()

    @pl.when(s < NT - 1)
    def _():
        pltpu.sync_copy(accd.at[pl.ds(r0, RPT_A)], degp_hbm.at[c, pl.ds(r0, RPT_A)])

    @pl.when(s == NT - 1)
    def _():
        pltpu.sync_copy(accd.at[pl.ds(r0, RPT_B)], degp_hbm.at[c, pl.ds(r0, RPT_B)])


@functools.partial(
    pl.kernel,
    mesh=_sc_mesh,
    out_type=jax.ShapeDtypeStruct((2 * N, FH), jnp.float32),
    scratch_types=[
        pltpu.VMEM((EPT,), jnp.int32),
        pltpu.VMEM((NCHUNK, CH), jnp.int32),
        pltpu.VMEM((CH, FH), jnp.float32),
        pltpu.VMEM((CH, FH), jnp.float32),
        pltpu.VMEM_SHARED((NPAD, FH), jnp.float32),
        pltpu.SemaphoreType.DMA,
        pltpu.SemaphoreType.DMA,
        pltpu.SemaphoreType.DMA,
        pltpu.SemaphoreType.DMA,
    ],
)
def _agg_sc(xs_hbm, srcadj_hbm, dst3_hbm, out_hbm, srca, dsta,
            gbuf0, gbuf1, acc, gsem0, gsem1, ssem0, ssem1):
    c = lax.axis_index("c")
    s = lax.axis_index("s")
    off = c * N
    r0 = s * RPT_A

    @pl.when(s < NT - 1)
    def _():
        pltpu.sync_copy(xs_hbm.at[pl.ds(off + r0, RPT_A)], acc.at[pl.ds(r0, RPT_A)])

    @pl.when(s == NT - 1)
    def _():
        pltpu.sync_copy(xs_hbm.at[pl.ds(off + r0, RPT_B)], acc.at[pl.ds(r0, RPT_B)])

    pltpu.sync_copy(srcadj_hbm.at[c, s], srca)
    pltpu.sync_copy(dst3_hbm.at[s], dsta)

    def start_gather(k, gbuf, gsem):
        pltpu.async_copy(xs_hbm.at[srca.at[pl.ds(k * CH, CH)]], gbuf, gsem)

    def wait_gather(gbuf, gsem):
        pltpu.make_async_copy(xs_hbm.at[srca.at[pl.ds(0, CH)]], gbuf, gsem).wait()

    def start_scatter(k, gbuf, ssem):
        pltpu.async_copy(gbuf, acc.at[dsta.at[k]], ssem, add=True)

    def wait_scatter(gbuf, ssem):
        pltpu.make_async_copy(gbuf, acc.at[dsta.at[0]], ssem).wait()

    start_gather(0, gbuf0, gsem0)
    plsc.subcore_barrier()

    def pair(k2, carry):
        ka = 2 * k2

        @pl.when(k2 > 0)
        def _():
            wait_scatter(gbuf1, ssem1)

        start_gather(ka + 1, gbuf1, gsem1)
        wait_gather(gbuf0, gsem0)
        start_scatter(ka, gbuf0, ssem0)

        @pl.when(k2 < NCHUNK // 2 - 1)
        def _():
            wait_scatter(gbuf0, ssem0)
            start_gather(ka + 2, gbuf0, gsem0)

        wait_gather(gbuf1, gsem1)
        start_scatter(ka + 1, gbuf1, ssem1)
        return carry

    lax.fori_loop(0, NCHUNK // 2, pair, 0)
    wait_scatter(gbuf0, ssem0)
    wait_scatter(gbuf1, ssem1)
    plsc.subcore_barrier()

    @pl.when(s < NT - 1)
    def _():
        pltpu.sync_copy(acc.at[pl.ds(r0, RPT_A)], out_hbm.at[pl.ds(off + r0, RPT_A)])

    @pl.when(s == NT - 1)
    def _():
        pltpu.sync_copy(acc.at[pl.ds(r0, RPT_B)], out_hbm.at[pl.ds(off + r0, RPT_B)])


def _scale_body(x_ref, degp_ref, xs_ref, normc_ref):
    d = degp_ref[0, :, 0:1] + degp_ref[1, :, 0:1]
    norm = lax.rsqrt(1.0 + d)
    normc_ref[...] = jnp.broadcast_to(norm, normc_ref.shape)
    xs = x_ref[...] * norm
    xs_ref[0] = xs[:, :FH]
    xs_ref[1] = xs[:, FH:]


def _mm_body(agg_ref, normc_ref, w1_ref, b1_ref, w2_ref, ps_ref):
    nc = normc_ref[...]
    a = jnp.concatenate([agg_ref[0] * nc, agg_ref[1] * nc], axis=1)
    h = jnp.dot(a, w1_ref[...], preferred_element_type=jnp.float32)
    h = jnp.maximum(h + b1_ref[...], 0.0)
    p = jnp.dot(h, w2_ref[...], preferred_element_type=jnp.float32)
    ps_ref[0] = p[:, :FH] * nc
    ps_ref[1] = p[:, FH:] * nc


def _final_body(agg_ref, normc_ref, b2_ref, out_ref):
    nc = normc_ref[...]
    out_ref[...] = jnp.concatenate(
        [agg_ref[0] * nc, agg_ref[1] * nc], axis=1) + b2_ref[...]


_BN = 1000
_G = N // _BN

_scale_call = pl.pallas_call(
    _scale_body,
    grid=(_G,),
    in_specs=[
        pl.BlockSpec((_BN, F_IN), lambda i: (i, 0)),
        pl.BlockSpec((2, _BN, FH), lambda i: (0, i, 0)),
    ],
    out_specs=[
        pl.BlockSpec((2, _BN, FH), lambda i: (0, i, 0)),
        pl.BlockSpec((_BN, FH), lambda i: (i, 0)),
    ],
    out_shape=[
        jax.ShapeDtypeStruct((2, N, FH), jnp.float32),
        jax.ShapeDtypeStruct((N, FH), jnp.float32),
    ],
)

_mm_call = pl.pallas_call(
    _mm_body,
    grid=(_G,),
    in_specs=[
        pl.BlockSpec((2, _BN, FH), lambda i: (0, i, 0)),
        pl.BlockSpec((_BN, FH), lambda i: (i, 0)),
        pl.BlockSpec((F_IN, HID), lambda i: (0, 0)),
        pl.BlockSpec((1, HID), lambda i: (0, 0)),
        pl.BlockSpec((HID, F_OUT), lambda i: (0, 0)),
    ],
    out_specs=pl.BlockSpec((2, _BN, FH), lambda i: (0, i, 0)),
    out_shape=jax.ShapeDtypeStruct((2, N, FH), jnp.float32),
)

_final_call = pl.pallas_call(
    _final_body,
    grid=(_G,),
    in_specs=[
        pl.BlockSpec((2, _BN, FH), lambda i: (0, i, 0)),
        pl.BlockSpec((_BN, FH), lambda i: (i, 0)),
        pl.BlockSpec((1, F_OUT), lambda i: (0, 0)),
    ],
    out_specs=pl.BlockSpec((_BN, F_OUT), lambda i: (i, 0)),
    out_shape=jax.ShapeDtypeStruct((N, F_OUT), jnp.float32),
)


def kernel(inputs, edge_index, W1, b1, W2, b2):
    src = edge_index[0].astype(jnp.int32)
    dst = edge_index[1].astype(jnp.int32)

    npad = EPT - E // NT
    pad_src = (jnp.arange(npad * NT, dtype=jnp.int32) * 401) % N
    pad_dst = N + (jnp.arange(npad * NT, dtype=jnp.int32) % 8)
    src_p = jnp.concatenate(
        [src.reshape(NT, E // NT), pad_src.reshape(NT, npad)],
        axis=1).reshape(NT, NCHUNK, CH)
    dst_p = jnp.concatenate(
        [dst.reshape(NT, E // NT), pad_dst.reshape(NT, npad)],
        axis=1).reshape(NT, NCHUNK, CH)
    src_adj = jnp.stack([src_p, src_p + N]).reshape(2, NT, EPT)

    zerosw = jnp.zeros((N, FH), jnp.float32)
    onesw = jnp.ones((CH, FH), jnp.float32)

    degp = _deg_sc(dst_p, zerosw, onesw)
    xs2, normc = _scale_call(inputs, degp)
    agg1 = _agg_sc(xs2.reshape(2 * N, FH), src_adj, dst_p)
    ps2 = _mm_call(agg1.reshape(2, N, FH), normc, W1,
                   b1.reshape(1, HID), W2)
    agg2 = _agg_sc(ps2.reshape(2 * N, FH), src_adj, dst_p)
    out = _final_call(agg2.reshape(2, N, FH), normc,
                      b2.reshape(1, F_OUT))
    return out


# speedup vs baseline: 14.0479x; 1.0108x over previous
import functools

import jax
import jax.numpy as jnp
from jax import lax
from jax.experimental import pallas as pl
from jax.experimental.pallas import tpu as pltpu
from jax.experimental.pallas import tpu_sc as plsc

N = 10000
E = 160000
F_IN = 256
HID = 512
F_OUT = 256
FH = 128

NT = 16
EPT = E // NT
CH = 80
NCHUNK = EPT // CH
DEG_C0 = 64
RPT_A = 632
RPT_B = N - 15 * RPT_A

_sc_mesh = plsc.VectorSubcoreMesh(core_axis_name="c", subcore_axis_name="s")


@functools.partial(
    pl.kernel,
    mesh=_sc_mesh,
    out_type=jax.ShapeDtypeStruct((2, N, FH), jnp.float32),
    scratch_types=[
        pltpu.VMEM((DEG_C0, CH), jnp.int32),
        pltpu.VMEM((CH, FH), jnp.float32),
        pltpu.VMEM_SHARED((N, FH), jnp.float32),
        pltpu.SemaphoreType.DMA,
    ],
)
def _deg_sc(dst3_hbm, zeros_hbm, ones_hbm, degp_hbm, dstd, onesb, accd, sem):
    c = lax.axis_index("c")
    s = lax.axis_index("s")
    r0 = s * RPT_A
    nch = DEG_C0 - 3 * c

    @pl.when(s < NT - 1)
    def _():
        pltpu.sync_copy(zeros_hbm.at[pl.ds(r0, RPT_A)], accd.at[pl.ds(r0, RPT_A)])

    @pl.when(s == NT - 1)
    def _():
        pltpu.sync_copy(zeros_hbm.at[pl.ds(r0, RPT_B)], accd.at[pl.ds(r0, RPT_B)])

    @pl.when(c == 0)
    def _():
        pltpu.sync_copy(dst3_hbm.at[s, pl.ds(0, DEG_C0)], dstd)

    @pl.when(c == 1)
    def _():
        pltpu.sync_copy(dst3_hbm.at[s, pl.ds(DEG_C0, NCHUNK - DEG_C0)],
                        dstd.at[pl.ds(0, NCHUNK - DEG_C0)])

    pltpu.sync_copy(ones_hbm, onesb)
    plsc.subcore_barrier()

    def fire(k, carry):
        pltpu.async_copy(onesb, accd.at[dstd.at[k]], sem, add=True)
        return carry

    lax.fori_loop(0, nch, fire, 0)

    def drain(k, carry):
        pltpu.make_async_copy(onesb, accd.at[dstd.at[0]], sem).wait()
        return carry

    lax.fori_loop(0, nch, drain, 0)
    plsc.subcore_barrier()

    @pl.when(s < NT - 1)
    def _():
        pltpu.sync_copy(accd.at[pl.ds(r0, RPT_A)], degp_hbm.at[c, pl.ds(r0, RPT_A)])

    @pl.when(s == NT - 1)
    def _():
        pltpu.sync_copy(accd.at[pl.ds(r0, RPT_B)], degp_hbm.at[c, pl.ds(r0, RPT_B)])


@functools.partial(
    pl.kernel,
    mesh=_sc_mesh,
    out_type=jax.ShapeDtypeStruct((2 * N, FH), jnp.float32),
    scratch_types=[
        pltpu.VMEM((EPT,), jnp.int32),
        pltpu.VMEM((NCHUNK, CH), jnp.int32),
        pltpu.VMEM((CH, FH), jnp.float32),
        pltpu.VMEM((CH, FH), jnp.float32),
        pltpu.VMEM_SHARED((N, FH), jnp.float32),
        pltpu.SemaphoreType.DMA,
        pltpu.SemaphoreType.DMA,
        pltpu.SemaphoreType.DMA,
        pltpu.SemaphoreType.DMA,
    ],
)
def _agg_sc(xs_hbm, srcadj_hbm, dst3_hbm, out_hbm, srca, dsta,
            gbuf0, gbuf1, acc, gsem0, gsem1, ssem0, ssem1):
    c = lax.axis_index("c")
    s = lax.axis_index("s")
    off = c * N
    r0 = s * RPT_A

    @pl.when(s < NT - 1)
    def _():
        pltpu.sync_copy(xs_hbm.at[pl.ds(off + r0, RPT_A)], acc.at[pl.ds(r0, RPT_A)])

    @pl.when(s == NT - 1)
    def _():
        pltpu.sync_copy(xs_hbm.at[pl.ds(off + r0, RPT_B)], acc.at[pl.ds(r0, RPT_B)])

    pltpu.sync_copy(srcadj_hbm.at[c, s], srca)
    pltpu.sync_copy(dst3_hbm.at[s], dsta)

    gbufs = (gbuf0, gbuf1)
    gsems = (gsem0, gsem1)
    ssems = (ssem0, ssem1)

    def start_gather(k, b):
        pltpu.async_copy(xs_hbm.at[srca.at[pl.ds(k * CH, CH)]], gbufs[b], gsems[b])

    def wait_gather(b):
        pltpu.make_async_copy(
            xs_hbm.at[srca.at[pl.ds(0, CH)]], gbufs[b], gsems[b]).wait()

    def start_scatter(k, b):
        pltpu.async_copy(gbufs[b], acc.at[dsta.at[k]], ssems[b], add=True)

    def wait_scatter(b):
        pltpu.make_async_copy(gbufs[b], acc.at[dsta.at[0]], ssems[b]).wait()

    start_gather(0, 0)
    plsc.subcore_barrier()

    def step(k, carry):
        par = k % 2
        for b in (0, 1):
            nb = 1 - b

            @pl.when(par == b)
            def _():
                @pl.when(k >= 1)
                def _():
                    wait_scatter(nb)

                @pl.when(k + 1 < NCHUNK)
                def _():
                    start_gather(k + 1, nb)

                wait_gather(b)
                start_scatter(k, b)

        return carry

    lax.fori_loop(0, NCHUNK, step, 0)
    wait_scatter((NCHUNK - 1) % 2)
    plsc.subcore_barrier()

    @pl.when(s < NT - 1)
    def _():
        pltpu.sync_copy(acc.at[pl.ds(r0, RPT_A)], out_hbm.at[pl.ds(off + r0, RPT_A)])

    @pl.when(s == NT - 1)
    def _():
        pltpu.sync_copy(acc.at[pl.ds(r0, RPT_B)], out_hbm.at[pl.ds(off + r0, RPT_B)])


def _scale_body(x_ref, degp_ref, xs_ref, normc_ref):
    d = degp_ref[0, :, 0:1] + degp_ref[1, :, 0:1]
    norm = lax.rsqrt(1.0 + d)
    normc_ref[...] = jnp.broadcast_to(norm, normc_ref.shape)
    xs = x_ref[...] * norm
    xs_ref[0] = xs[:, :FH]
    xs_ref[1] = xs[:, FH:]


def _mm_body(agg_ref, normc_ref, w1_ref, b1_ref, w2_ref, ps_ref):
    nc = normc_ref[...]
    a = jnp.concatenate([agg_ref[0] * nc, agg_ref[1] * nc], axis=1)
    h = jnp.dot(a, w1_ref[...], preferred_element_type=jnp.float32)
    h = jnp.maximum(h + b1_ref[...], 0.0)
    p = jnp.dot(h, w2_ref[...], preferred_element_type=jnp.float32)
    ps_ref[0] = p[:, :FH] * nc
    ps_ref[1] = p[:, FH:] * nc


def _final_body(agg_ref, normc_ref, b2_ref, out_ref):
    nc = normc_ref[...]
    out_ref[...] = jnp.concatenate(
        [agg_ref[0] * nc, agg_ref[1] * nc], axis=1) + b2_ref[...]


_BN = 1000
_G = N // _BN

_scale_call = pl.pallas_call(
    _scale_body,
    grid=(_G,),
    in_specs=[
        pl.BlockSpec((_BN, F_IN), lambda i: (i, 0)),
        pl.BlockSpec((2, _BN, FH), lambda i: (0, i, 0)),
    ],
    out_specs=[
        pl.BlockSpec((2, _BN, FH), lambda i: (0, i, 0)),
        pl.BlockSpec((_BN, FH), lambda i: (i, 0)),
    ],
    out_shape=[
        jax.ShapeDtypeStruct((2, N, FH), jnp.float32),
        jax.ShapeDtypeStruct((N, FH), jnp.float32),
    ],
)

_mm_call = pl.pallas_call(
    _mm_body,
    grid=(_G,),
    in_specs=[
        pl.BlockSpec((2, _BN, FH), lambda i: (0, i, 0)),
        pl.BlockSpec((_BN, FH), lambda i: (i, 0)),
        pl.BlockSpec((F_IN, HID), lambda i: (0, 0)),
        pl.BlockSpec((1, HID), lambda i: (0, 0)),
        pl.BlockSpec((HID, F_OUT), lambda i: (0, 0)),
    ],
    out_specs=pl.BlockSpec((2, _BN, FH), lambda i: (0, i, 0)),
    out_shape=jax.ShapeDtypeStruct((2, N, FH), jnp.float32),
)

_final_call = pl.pallas_call(
    _final_body,
    grid=(_G,),
    in_specs=[
        pl.BlockSpec((2, _BN, FH), lambda i: (0, i, 0)),
        pl.BlockSpec((_BN, FH), lambda i: (i, 0)),
        pl.BlockSpec((1, F_OUT), lambda i: (0, 0)),
    ],
    out_specs=pl.BlockSpec((_BN, F_OUT), lambda i: (i, 0)),
    out_shape=jax.ShapeDtypeStruct((N, F_OUT), jnp.float32),
)


def kernel(inputs, edge_index, W1, b1, W2, b2):
    src = edge_index[0].astype(jnp.int32)
    dst = edge_index[1].astype(jnp.int32)

    src_p = src.reshape(NT, EPT)
    dst_p = dst.reshape(NT, NCHUNK, CH)
    src_adj = jnp.stack([src_p, src_p + N])

    zerosw = jnp.zeros((N, FH), jnp.float32)
    onesw = jnp.ones((CH, FH), jnp.float32)

    degp = _deg_sc(dst_p, zerosw, onesw)
    xs2, normc = _scale_call(inputs, degp)
    agg1 = _agg_sc(xs2.reshape(2 * N, FH), src_adj, dst_p)
    ps2 = _mm_call(agg1.reshape(2, N, FH), normc, W1,
                   b1.reshape(1, HID), W2)
    agg2 = _agg_sc(ps2.reshape(2 * N, FH), src_adj, dst_p)
    out = _final_call(agg2.reshape(2, N, FH), normc,
                      b2.reshape(1, F_OUT))
    return out


# speedup vs baseline: 15.4607x; 1.1006x over previous
import functools

import jax
import jax.numpy as jnp
from jax import lax
from jax.experimental import pallas as pl
from jax.experimental.pallas import tpu as pltpu
from jax.experimental.pallas import tpu_sc as plsc

N = 10000
E = 160000
F_IN = 256
HID = 512
F_OUT = 256
FH = 128

NT = 16
EPT = E // NT
CH = 80
NCHUNK = EPT // CH
DEG_C0 = 64
RPT_A = 632
RPT_B = N - 15 * RPT_A

_sc_mesh = plsc.VectorSubcoreMesh(core_axis_name="c", subcore_axis_name="s")


@functools.partial(
    pl.kernel,
    mesh=_sc_mesh,
    out_type=jax.ShapeDtypeStruct((2, N, FH), jnp.float32),
    scratch_types=[
        pltpu.VMEM((DEG_C0, CH), jnp.int32),
        pltpu.VMEM((CH, FH), jnp.float32),
        pltpu.VMEM_SHARED((N, FH), jnp.float32),
        pltpu.SemaphoreType.DMA,
    ],
)
def _deg_sc(dst3_hbm, zeros_hbm, ones_hbm, degp_hbm, dstd, onesb, accd, sem):
    c = lax.axis_index("c")
    s = lax.axis_index("s")
    r0 = s * RPT_A
    nch = DEG_C0 - 3 * c

    @pl.when(s < NT - 1)
    def _():
        pltpu.sync_copy(zeros_hbm.at[pl.ds(r0, RPT_A)], accd.at[pl.ds(r0, RPT_A)])

    @pl.when(s == NT - 1)
    def _():
        pltpu.sync_copy(zeros_hbm.at[pl.ds(r0, RPT_B)], accd.at[pl.ds(r0, RPT_B)])

    @pl.when(c == 0)
    def _():
        pltpu.sync_copy(dst3_hbm.at[s, pl.ds(0, DEG_C0)], dstd)

    @pl.when(c == 1)
    def _():
        pltpu.sync_copy(dst3_hbm.at[s, pl.ds(DEG_C0, NCHUNK - DEG_C0)],
                        dstd.at[pl.ds(0, NCHUNK - DEG_C0)])

    pltpu.sync_copy(ones_hbm, onesb)
    plsc.subcore_barrier()

    def fire(k, carry):
        pltpu.async_copy(onesb, accd.at[dstd.at[k]], sem, add=True)
        return carry

    lax.fori_loop(0, nch, fire, 0)

    def drain(k, carry):
        pltpu.make_async_copy(onesb, accd.at[dstd.at[0]], sem).wait()
        return carry

    lax.fori_loop(0, nch, drain, 0)
    plsc.subcore_barrier()

    @pl.when(s < NT - 1)
    def _():
        pltpu.sync_copy(accd.at[pl.ds(r0, RPT_A)], degp_hbm.at[c, pl.ds(r0, RPT_A)])

    @pl.when(s == NT - 1)
    def _():
        pltpu.sync_copy(accd.at[pl.ds(r0, RPT_B)], degp_hbm.at[c, pl.ds(r0, RPT_B)])


@functools.partial(
    pl.kernel,
    mesh=_sc_mesh,
    out_type=jax.ShapeDtypeStruct((2 * N, FH), jnp.float32),
    scratch_types=[
        pltpu.VMEM((EPT,), jnp.int32),
        pltpu.VMEM((NCHUNK, CH), jnp.int32),
        pltpu.VMEM((CH, FH), jnp.float32),
        pltpu.VMEM((CH, FH), jnp.float32),
        pltpu.VMEM_SHARED((N, FH), jnp.float32),
        pltpu.SemaphoreType.DMA,
        pltpu.SemaphoreType.DMA,
        pltpu.SemaphoreType.DMA,
        pltpu.SemaphoreType.DMA,
    ],
)
def _agg_sc(xs_hbm, srcadj_hbm, dst3_hbm, out_hbm, srca, dsta,
            gbuf0, gbuf1, acc, gsem0, gsem1, ssem0, ssem1):
    c = lax.axis_index("c")
    s = lax.axis_index("s")
    off = c * N
    r0 = s * RPT_A

    @pl.when(s < NT - 1)
    def _():
        pltpu.sync_copy(xs_hbm.at[pl.ds(off + r0, RPT_A)], acc.at[pl.ds(r0, RPT_A)])

    @pl.when(s == NT - 1)
    def _():
        pltpu.sync_copy(xs_hbm.at[pl.ds(off + r0, RPT_B)], acc.at[pl.ds(r0, RPT_B)])

    pltpu.sync_copy(srcadj_hbm.at[c, s], srca)
    pltpu.sync_copy(dst3_hbm.at[s], dsta)

    gbufs = (gbuf0, gbuf1)
    gsems = (gsem0, gsem1)
    ssems = (ssem0, ssem1)

    def start_gather(k, b):
        pltpu.async_copy(xs_hbm.at[srca.at[pl.ds(k * CH, CH)]], gbufs[b], gsems[b])

    def wait_gather(b):
        pltpu.make_async_copy(
            xs_hbm.at[srca.at[pl.ds(0, CH)]], gbufs[b], gsems[b]).wait()

    def start_scatter(k, b):
        pltpu.async_copy(gbufs[b], acc.at[dsta.at[k]], ssems[b], add=True)

    def wait_scatter(b):
        pltpu.make_async_copy(gbufs[b], acc.at[dsta.at[0]], ssems[b]).wait()

    start_gather(0, 0)
    plsc.subcore_barrier()

    def step(k, carry):
        par = k % 2
        for b in (0, 1):
            nb = 1 - b

            @pl.when(par == b)
            def _():
                @pl.when(k + 1 < NCHUNK)
                def _():
                    start_gather(k + 1, nb)

                wait_gather(b)

        return carry

    lax.fori_loop(0, NCHUNK, step, 0)
    plsc.subcore_barrier()

    @pl.when(s < NT - 1)
    def _():
        pltpu.sync_copy(acc.at[pl.ds(r0, RPT_A)], out_hbm.at[pl.ds(off + r0, RPT_A)])

    @pl.when(s == NT - 1)
    def _():
        pltpu.sync_copy(acc.at[pl.ds(r0, RPT_B)], out_hbm.at[pl.ds(off + r0, RPT_B)])


def _scale_body(x_ref, degp_ref, xs_ref, normc_ref):
    d = degp_ref[0, :, 0:1] + degp_ref[1, :, 0:1]
    norm = lax.rsqrt(1.0 + d)
    normc_ref[...] = jnp.broadcast_to(norm, normc_ref.shape)
    xs = x_ref[...] * norm
    xs_ref[0] = xs[:, :FH]
    xs_ref[1] = xs[:, FH:]


def _mm_body(agg_ref, normc_ref, w1_ref, b1_ref, w2_ref, ps_ref):
    nc = normc_ref[...]
    a = jnp.concatenate([agg_ref[0] * nc, agg_ref[1] * nc], axis=1)
    h = jnp.dot(a, w1_ref[...], preferred_element_type=jnp.float32)
    h = jnp.maximum(h + b1_ref[...], 0.0)
    p = jnp.dot(h, w2_ref[...], preferred_element_type=jnp.float32)
    ps_ref[0] = p[:, :FH] * nc
    ps_ref[1] = p[:, FH:] * nc


def _final_body(agg_ref, normc_ref, b2_ref, out_ref):
    nc = normc_ref[...]
    out_ref[...] = jnp.concatenate(
        [agg_ref[0] * nc, agg_ref[1] * nc], axis=1) + b2_ref[...]


_BN = 1000
_G = N // _BN

_scale_call = pl.pallas_call(
    _scale_body,
    grid=(_G,),
    in_specs=[
        pl.BlockSpec((_BN, F_IN), lambda i: (i, 0)),
        pl.BlockSpec((2, _BN, FH), lambda i: (0, i, 0)),
    ],
    out_specs=[
        pl.BlockSpec((2, _BN, FH), lambda i: (0, i, 0)),
        pl.BlockSpec((_BN, FH), lambda i: (i, 0)),
    ],
    out_shape=[
        jax.ShapeDtypeStruct((2, N, FH), jnp.float32),
        jax.ShapeDtypeStruct((N, FH), jnp.float32),
    ],
)

_mm_call = pl.pallas_call(
    _mm_body,
    grid=(_G,),
    in_specs=[
        pl.BlockSpec((2, _BN, FH), lambda i: (0, i, 0)),
        pl.BlockSpec((_BN, FH), lambda i: (i, 0)),
        pl.BlockSpec((F_IN, HID), lambda i: (0, 0)),
        pl.BlockSpec((1, HID), lambda i: (0, 0)),
        pl.BlockSpec((HID, F_OUT), lambda i: (0, 0)),
    ],
    out_specs=pl.BlockSpec((2, _BN, FH), lambda i: (0, i, 0)),
    out_shape=jax.ShapeDtypeStruct((2, N, FH), jnp.float32),
)

_final_call = pl.pallas_call(
    _final_body,
    grid=(_G,),
    in_specs=[
        pl.BlockSpec((2, _BN, FH), lambda i: (0, i, 0)),
        pl.BlockSpec((_BN, FH), lambda i: (i, 0)),
        pl.BlockSpec((1, F_OUT), lambda i: (0, 0)),
    ],
    out_specs=pl.BlockSpec((_BN, F_OUT), lambda i: (i, 0)),
    out_shape=jax.ShapeDtypeStruct((N, F_OUT), jnp.float32),
)


def kernel(inputs, edge_index, W1, b1, W2, b2):
    src = edge_index[0].astype(jnp.int32)
    dst = edge_index[1].astype(jnp.int32)

    src_p = src.reshape(NT, EPT)
    dst_p = dst.reshape(NT, NCHUNK, CH)
    src_adj = jnp.stack([src_p, src_p + N])

    zerosw = jnp.zeros((N, FH), jnp.float32)
    onesw = jnp.ones((CH, FH), jnp.float32)

    degp = _deg_sc(dst_p, zerosw, onesw)
    xs2, normc = _scale_call(inputs, degp)
    agg1 = _agg_sc(xs2.reshape(2 * N, FH), src_adj, dst_p)
    ps2 = _mm_call(agg1.reshape(2, N, FH), normc, W1,
                   b1.reshape(1, HID), W2)
    agg2 = _agg_sc(ps2.reshape(2 * N, FH), src_adj, dst_p)
    out = _final_call(agg2.reshape(2, N, FH), normc,
                      b2.reshape(1, F_OUT))
    return out


# speedup vs baseline: 15.6331x; 1.0111x over previous
import functools

import jax
import jax.numpy as jnp
from jax import lax
from jax.experimental import pallas as pl
from jax.experimental.pallas import tpu as pltpu
from jax.experimental.pallas import tpu_sc as plsc

N = 10000
E = 160000
F_IN = 256
HID = 512
F_OUT = 256
FH = 128

NT = 16
EPT = E // NT
CH = 80
NCHUNK = EPT // CH
DEG_C0 = 64
NB = 16
NCHB = 128
RPT_A = 632
RPT_B = N - 15 * RPT_A

_sc_mesh = plsc.VectorSubcoreMesh(core_axis_name="c", subcore_axis_name="s")


@functools.partial(
    pl.kernel,
    mesh=_sc_mesh,
    out_type=jax.ShapeDtypeStruct((2, N, FH), jnp.float32),
    scratch_types=[
        pltpu.VMEM((DEG_C0, CH), jnp.int32),
        pltpu.VMEM((CH, FH), jnp.float32),
        pltpu.VMEM_SHARED((N, FH), jnp.float32),
        pltpu.SemaphoreType.DMA,
    ],
)
def _deg_sc(dst3_hbm, zeros_hbm, ones_hbm, degp_hbm, dstd, onesb, accd, sem):
    c = lax.axis_index("c")
    s = lax.axis_index("s")
    r0 = s * RPT_A
    nch = DEG_C0 - 3 * c

    @pl.when(s < NT - 1)
    def _():
        pltpu.sync_copy(zeros_hbm.at[pl.ds(r0, RPT_A)], accd.at[pl.ds(r0, RPT_A)])

    @pl.when(s == NT - 1)
    def _():
        pltpu.sync_copy(zeros_hbm.at[pl.ds(r0, RPT_B)], accd.at[pl.ds(r0, RPT_B)])

    pltpu.sync_copy(dst3_hbm.at[s, pl.ds(c * DEG_C0, DEG_C0)], dstd)
    pltpu.sync_copy(ones_hbm, onesb)
    plsc.subcore_barrier()

    def fire(k, carry):
        pltpu.async_copy(onesb, accd.at[dstd.at[k]], sem, add=True)
        return carry

    lax.fori_loop(0, nch, fire, 0)

    def drain(k, carry):
        pltpu.make_async_copy(onesb, accd.at[dstd.at[0]], sem).wait()
        return carry

    lax.fori_loop(0, nch, drain, 0)
    plsc.subcore_barrier()

    @pl.when(s < NT - 1)
    def _():
        pltpu.sync_copy(accd.at[pl.ds(r0, RPT_A)], degp_hbm.at[c, pl.ds(r0, RPT_A)])

    @pl.when(s == NT - 1)
    def _():
        pltpu.sync_copy(accd.at[pl.ds(r0, RPT_B)], degp_hbm.at[c, pl.ds(r0, RPT_B)])


@functools.partial(
    pl.kernel,
    mesh=_sc_mesh,
    out_type=jax.ShapeDtypeStruct((2 * N, FH), jnp.float32),
    scratch_types=[
        pltpu.VMEM((NB, CH), jnp.int32),
        pltpu.VMEM((NB, CH), jnp.int32),
        pltpu.VMEM((NB, CH), jnp.int32),
        pltpu.VMEM((NB, CH), jnp.int32),
        pltpu.VMEM((CH, FH), jnp.float32),
        pltpu.VMEM((CH, FH), jnp.float32),
        pltpu.VMEM((CH, FH), jnp.float32),
        pltpu.VMEM((CH, FH), jnp.float32),
        pltpu.VMEM_SHARED((N, FH), jnp.float32),
        pltpu.SemaphoreType.DMA,
        pltpu.SemaphoreType.DMA,
        pltpu.SemaphoreType.DMA,
        pltpu.SemaphoreType.DMA,
        pltpu.SemaphoreType.DMA,
        pltpu.SemaphoreType.DMA,
        pltpu.SemaphoreType.DMA,
        pltpu.SemaphoreType.DMA,
        pltpu.SemaphoreType.DMA,
        pltpu.SemaphoreType.DMA,
        pltpu.SemaphoreType.DMA,
        pltpu.SemaphoreType.DMA,
    ],
)
def _agg_sc(xs_hbm, srcadj_hbm, dst3_hbm, out_hbm,
            srcb0, srcb1, dstb0, dstb1, gbuf0, gbuf1, gbuf2, gbuf3, acc,
            gsem0, gsem1, gsem2, gsem3, ssem0, ssem1, ssem2, ssem3,
            bsem0, bsem1, dsem0, dsem1):
    c = lax.axis_index("c")
    s = lax.axis_index("s")
    off = c * N
    r0 = s * RPT_A

    @pl.when(s < NT - 1)
    def _():
        pltpu.sync_copy(xs_hbm.at[pl.ds(off + r0, RPT_A)], acc.at[pl.ds(r0, RPT_A)])

    @pl.when(s == NT - 1)
    def _():
        pltpu.sync_copy(xs_hbm.at[pl.ds(off + r0, RPT_B)], acc.at[pl.ds(r0, RPT_B)])

    gbufs = (gbuf0, gbuf1, gbuf2, gbuf3)
    gsems = (gsem0, gsem1, gsem2, gsem3)
    ssems = (ssem0, ssem1, ssem2, ssem3)
    srcbs = (srcb0, srcb1)
    dstbs = (dstb0, dstb1)
    bsems = (bsem0, bsem1)
    dsems = (dsem0, dsem1)

    def start_batch(bt, r):
        pltpu.async_copy(srcadj_hbm.at[c, s, pl.ds(bt * NB, NB)], srcbs[r], bsems[r])
        pltpu.async_copy(dst3_hbm.at[s, pl.ds(bt * NB, NB)], dstbs[r], dsems[r])

    def wait_batch(r):
        pltpu.make_async_copy(
            srcadj_hbm.at[c, s, pl.ds(0, NB)], srcbs[r], bsems[r]).wait()
        pltpu.make_async_copy(
            dst3_hbm.at[s, pl.ds(0, NB)], dstbs[r], dsems[r]).wait()

    def start_gather(k, b):
        row = lax.rem(k, NB)
        rr = lax.rem(k // NB, 2)
        for r in (0, 1):
            @pl.when(rr == r)
            def _(r=r):
                pltpu.async_copy(xs_hbm.at[srcbs[r].at[row]], gbufs[b], gsems[b])

    def wait_gather(b):
        pltpu.make_async_copy(xs_hbm.at[srcbs[0].at[0]], gbufs[b], gsems[b]).wait()

    def start_scatter(k, b):
        row = lax.rem(k, NB)
        rr = lax.rem(k // NB, 2)
        for r in (0, 1):
            @pl.when(rr == r)
            def _(r=r):
                pltpu.async_copy(gbufs[b], acc.at[dstbs[r].at[row]], ssems[b], add=True)

    def wait_scatter(b):
        pltpu.make_async_copy(gbufs[b], acc.at[dstbs[0].at[0]], ssems[b]).wait()

    start_batch(0, 0)
    start_batch(1, 1)
    wait_batch(0)
    for _b in range(3):
        start_gather(_b, _b)
    plsc.subcore_barrier()

    def step(k, carry):
        par = k % 4

        bt = k // NB
        cond_a = (lax.rem(k, NB) == 3) & (k >= NB) & (bt + 1 <= (NCHUNK - 1) // NB)
        for r in (0, 1):
            @pl.when(cond_a & (lax.rem(bt + 1, 2) == r))
            def _(r=r):
                start_batch(bt + 1, r)

        cond_b = (k + 3 < NCHUNK) & (lax.rem(k + 3, NB) == 0)
        for r in (0, 1):
            @pl.when(cond_b & (lax.rem((k + 3) // NB, 2) == r))
            def _(r=r):
                wait_batch(r)

        for b in (0, 1, 2, 3):
            nb = (b + 3) % 4

            @pl.when(par == b)
            def _(b=b, nb=nb):
                wait_gather(b)
                start_scatter(k, b)

                @pl.when(k + 3 < NCHUNK)
                def _():
                    @pl.when(k >= 1)
                    def _():
                        wait_scatter(nb)

                    start_gather(k + 3, nb)

        return carry

    lax.fori_loop(0, NCHUNK, step, 0)
    wait_scatter((NCHUNK - 4) % 4)
    wait_scatter((NCHUNK - 3) % 4)
    wait_scatter((NCHUNK - 2) % 4)
    wait_scatter((NCHUNK - 1) % 4)
    plsc.subcore_barrier()

    @pl.when(s < NT - 1)
    def _():
        pltpu.sync_copy(acc.at[pl.ds(r0, RPT_A)], out_hbm.at[pl.ds(off + r0, RPT_A)])

    @pl.when(s == NT - 1)
    def _():
        pltpu.sync_copy(acc.at[pl.ds(r0, RPT_B)], out_hbm.at[pl.ds(off + r0, RPT_B)])


def _scale_body(x_ref, degp_ref, xs_ref, normc_ref):
    d = degp_ref[0, :, 0:1] + degp_ref[1, :, 0:1]
    norm = lax.rsqrt(1.0 + d)
    normc_ref[...] = jnp.broadcast_to(norm, normc_ref.shape)
    xs = x_ref[...] * norm
    xs_ref[0] = xs[:, :FH]
    xs_ref[1] = xs[:, FH:]


def _mm_body(agg_ref, normc_ref, w1_ref, b1_ref, w2_ref, ps_ref):
    nc = normc_ref[...]
    a = jnp.concatenate([agg_ref[0] * nc, agg_ref[1] * nc], axis=1)
    h = jnp.dot(a, w1_ref[...], preferred_element_type=jnp.float32)
    h = jnp.maximum(h + b1_ref[...], 0.0)
    p = jnp.dot(h, w2_ref[...], preferred_element_type=jnp.float32)
    ps_ref[0] = p[:, :FH] * nc
    ps_ref[1] = p[:, FH:] * nc


def _final_body(agg_ref, normc_ref, b2_ref, out_ref):
    nc = normc_ref[...]
    out_ref[...] = jnp.concatenate(
        [agg_ref[0] * nc, agg_ref[1] * nc], axis=1) + b2_ref[...]


_BN = 1000
_G = N // _BN

_scale_call = pl.pallas_call(
    _scale_body,
    grid=(_G,),
    in_specs=[
        pl.BlockSpec((_BN, F_IN), lambda i: (i, 0)),
        pl.BlockSpec((2, _BN, FH), lambda i: (0, i, 0)),
    ],
    out_specs=[
        pl.BlockSpec((2, _BN, FH), lambda i: (0, i, 0)),
        pl.BlockSpec((_BN, FH), lambda i: (i, 0)),
    ],
    out_shape=[
        jax.ShapeDtypeStruct((2, N, FH), jnp.float32),
        jax.ShapeDtypeStruct((N, FH), jnp.float32),
    ],
)

_mm_call = pl.pallas_call(
    _mm_body,
    grid=(_G,),
    in_specs=[
        pl.BlockSpec((2, _BN, FH), lambda i: (0, i, 0)),
        pl.BlockSpec((_BN, FH), lambda i: (i, 0)),
        pl.BlockSpec((F_IN, HID), lambda i: (0, 0)),
        pl.BlockSpec((1, HID), lambda i: (0, 0)),
        pl.BlockSpec((HID, F_OUT), lambda i: (0, 0)),
    ],
    out_specs=pl.BlockSpec((2, _BN, FH), lambda i: (0, i, 0)),
    out_shape=jax.ShapeDtypeStruct((2, N, FH), jnp.float32),
)

_final_call = pl.pallas_call(
    _final_body,
    grid=(_G,),
    in_specs=[
        pl.BlockSpec((2, _BN, FH), lambda i: (0, i, 0)),
        pl.BlockSpec((_BN, FH), lambda i: (i, 0)),
        pl.BlockSpec((1, F_OUT), lambda i: (0, 0)),
    ],
    out_specs=pl.BlockSpec((_BN, F_OUT), lambda i: (i, 0)),
    out_shape=jax.ShapeDtypeStruct((N, F_OUT), jnp.float32),
)


def kernel(inputs, edge_index, W1, b1, W2, b2):
    src = edge_index[0].astype(jnp.int32)
    dst = edge_index[1].astype(jnp.int32)

    src_p = src.reshape(NT, NCHUNK, CH)
    src_p = jnp.pad(src_p, ((0, 0), (0, NCHB - NCHUNK), (0, 0)))
    dst_p = dst.reshape(NT, NCHUNK, CH)
    dst_p = jnp.pad(dst_p, ((0, 0), (0, NCHB - NCHUNK), (0, 0)))
    src_adj = jnp.stack([src_p, src_p + N])

    zerosw = jnp.zeros((N, FH), jnp.float32)
    onesw = jnp.ones((CH, FH), jnp.float32)

    degp = _deg_sc(dst_p, zerosw, onesw)
    xs2, normc = _scale_call(inputs, degp)
    agg1 = _agg_sc(xs2.reshape(2 * N, FH), src_adj, dst_p)
    ps2 = _mm_call(agg1.reshape(2, N, FH), normc, W1,
                   b1.reshape(1, HID), W2)
    agg2 = _agg_sc(ps2.reshape(2 * N, FH), src_adj, dst_p)
    out = _final_call(agg2.reshape(2, N, FH), normc,
                      b2.reshape(1, F_OUT))
    return out


# speedup vs baseline: 15.6974x; 1.0041x over previous
import functools

import jax
import jax.numpy as jnp
from jax import lax
from jax.experimental import pallas as pl
from jax.experimental.pallas import tpu as pltpu
from jax.experimental.pallas import tpu_sc as plsc

N = 10000
E = 160000
F_IN = 256
HID = 512
F_OUT = 256
FH = 128

NT = 16
EPT = E // NT
CH = 80
NCHUNK = EPT // CH
DEG_C0 = 64
NB = 16
NCHB = 128
RPT_A = 632
RPT_B = N - 15 * RPT_A

_sc_mesh = plsc.VectorSubcoreMesh(core_axis_name="c", subcore_axis_name="s")


@functools.partial(
    pl.kernel,
    mesh=_sc_mesh,
    out_type=jax.ShapeDtypeStruct((2, N, FH), jnp.float32),
    scratch_types=[
        pltpu.VMEM((DEG_C0, CH), jnp.int32),
        pltpu.VMEM((CH, FH), jnp.float32),
        pltpu.VMEM_SHARED((N, FH), jnp.float32),
        pltpu.SemaphoreType.DMA,
    ],
)
def _deg_sc(dst3_hbm, zeros_hbm, ones_hbm, degp_hbm, dstd, onesb, accd, sem):
    c = lax.axis_index("c")
    s = lax.axis_index("s")
    r0 = s * RPT_A
    nch = DEG_C0 - 3 * c

    @pl.when(s < NT - 1)
    def _():
        pltpu.sync_copy(zeros_hbm.at[pl.ds(r0, RPT_A)], accd.at[pl.ds(r0, RPT_A)])

    @pl.when(s == NT - 1)
    def _():
        pltpu.sync_copy(zeros_hbm.at[pl.ds(r0, RPT_B)], accd.at[pl.ds(r0, RPT_B)])

    pltpu.sync_copy(dst3_hbm.at[s, pl.ds(c * DEG_C0, DEG_C0)], dstd)
    pltpu.sync_copy(ones_hbm, onesb)
    plsc.subcore_barrier()

    def fire(k, carry):
        @pl.when(k >= 8)
        def _():
            pltpu.make_async_copy(onesb, accd.at[dstd.at[0]], sem).wait()

        pltpu.async_copy(onesb, accd.at[dstd.at[k]], sem, add=True)
        return carry

    lax.fori_loop(0, nch, fire, 0)

    def drain(k, carry):
        pltpu.make_async_copy(onesb, accd.at[dstd.at[0]], sem).wait()
        return carry

    lax.fori_loop(0, 8, drain, 0)
    plsc.subcore_barrier()

    @pl.when(s < NT - 1)
    def _():
        pltpu.sync_copy(accd.at[pl.ds(r0, RPT_A)], degp_hbm.at[c, pl.ds(r0, RPT_A)])

    @pl.when(s == NT - 1)
    def _():
        pltpu.sync_copy(accd.at[pl.ds(r0, RPT_B)], degp_hbm.at[c, pl.ds(r0, RPT_B)])


@functools.partial(
    pl.kernel,
    mesh=_sc_mesh,
    out_type=jax.ShapeDtypeStruct((2 * N, FH), jnp.float32),
    scratch_types=[
        pltpu.VMEM((NB, CH), jnp.int32),
        pltpu.VMEM((NB, CH), jnp.int32),
        pltpu.VMEM((NB, CH), jnp.int32),
        pltpu.VMEM((NB, CH), jnp.int32),
        pltpu.VMEM((CH, FH), jnp.float32),
        pltpu.VMEM((CH, FH), jnp.float32),
        pltpu.VMEM((CH, FH), jnp.float32),
        pltpu.VMEM((CH, FH), jnp.float32),
        pltpu.VMEM_SHARED((N, FH), jnp.float32),
        pltpu.SemaphoreType.DMA,
        pltpu.SemaphoreType.DMA,
        pltpu.SemaphoreType.DMA,
        pltpu.SemaphoreType.DMA,
        pltpu.SemaphoreType.DMA,
        pltpu.SemaphoreType.DMA,
        pltpu.SemaphoreType.DMA,
        pltpu.SemaphoreType.DMA,
        pltpu.SemaphoreType.DMA,
        pltpu.SemaphoreType.DMA,
        pltpu.SemaphoreType.DMA,
        pltpu.SemaphoreType.DMA,
    ],
)
def _agg_sc(xs_hbm, srcadj_hbm, dst3_hbm, out_hbm,
            srcb0, srcb1, dstb0, dstb1, gbuf0, gbuf1, gbuf2, gbuf3, acc,
            gsem0, gsem1, gsem2, gsem3, ssem0, ssem1, ssem2, ssem3,
            bsem0, bsem1, dsem0, dsem1):
    c = lax.axis_index("c")
    s = lax.axis_index("s")
    off = c * N
    r0 = s * RPT_A

    @pl.when(s < NT - 1)
    def _():
        pltpu.sync_copy(xs_hbm.at[pl.ds(off + r0, RPT_A)], acc.at[pl.ds(r0, RPT_A)])

    @pl.when(s == NT - 1)
    def _():
        pltpu.sync_copy(xs_hbm.at[pl.ds(off + r0, RPT_B)], acc.at[pl.ds(r0, RPT_B)])

    gbufs = (gbuf0, gbuf1, gbuf2, gbuf3)
    gsems = (gsem0, gsem1, gsem2, gsem3)
    ssems = (ssem0, ssem1, ssem2, ssem3)
    srcbs = (srcb0, srcb1)
    dstbs = (dstb0, dstb1)
    bsems = (bsem0, bsem1)
    dsems = (dsem0, dsem1)

    def start_batch(bt, r):
        pltpu.async_copy(srcadj_hbm.at[c, s, pl.ds(bt * NB, NB)], srcbs[r], bsems[r])
        pltpu.async_copy(dst3_hbm.at[s, pl.ds(bt * NB, NB)], dstbs[r], dsems[r])

    def wait_batch(r):
        pltpu.make_async_copy(
            srcadj_hbm.at[c, s, pl.ds(0, NB)], srcbs[r], bsems[r]).wait()
        pltpu.make_async_copy(
            dst3_hbm.at[s, pl.ds(0, NB)], dstbs[r], dsems[r]).wait()

    def start_gather(k, b):
        row = lax.rem(k, NB)
        rr = lax.rem(k // NB, 2)
        for r in (0, 1):
            @pl.when(rr == r)
            def _(r=r):
                pltpu.async_copy(xs_hbm.at[srcbs[r].at[row]], gbufs[b], gsems[b])

    def wait_gather(b):
        pltpu.make_async_copy(xs_hbm.at[srcbs[0].at[0]], gbufs[b], gsems[b]).wait()

    def start_scatter(k, b):
        row = lax.rem(k, NB)
        rr = lax.rem(k // NB, 2)
        for r in (0, 1):
            @pl.when(rr == r)
            def _(r=r):
                pltpu.async_copy(gbufs[b], acc.at[dstbs[r].at[row]], ssems[b], add=True)

    def wait_scatter(b):
        pltpu.make_async_copy(gbufs[b], acc.at[dstbs[0].at[0]], ssems[b]).wait()

    start_batch(0, 0)
    start_batch(1, 1)
    wait_batch(0)
    for _b in range(3):
        start_gather(_b, _b)
    plsc.subcore_barrier()

    def step(k, carry):
        par = k % 4

        bt = k // NB
        cond_a = (lax.rem(k, NB) == 3) & (k >= NB) & (bt + 1 <= (NCHUNK - 1) // NB)
        for r in (0, 1):
            @pl.when(cond_a & (lax.rem(bt + 1, 2) == r))
            def _(r=r):
                start_batch(bt + 1, r)

        cond_b = (k + 3 < NCHUNK) & (lax.rem(k + 3, NB) == 0)
        for r in (0, 1):
            @pl.when(cond_b & (lax.rem((k + 3) // NB, 2) == r))
            def _(r=r):
                wait_batch(r)

        for b in (0, 1, 2, 3):
            nb = (b + 3) % 4

            @pl.when(par == b)
            def _(b=b, nb=nb):
                wait_gather(b)
                start_scatter(k, b)

                @pl.when(k + 3 < NCHUNK)
                def _():
                    @pl.when(k >= 1)
                    def _():
                        wait_scatter(nb)

                    start_gather(k + 3, nb)

        return carry

    lax.fori_loop(0, NCHUNK, step, 0)
    wait_scatter((NCHUNK - 4) % 4)
    wait_scatter((NCHUNK - 3) % 4)
    wait_scatter((NCHUNK - 2) % 4)
    wait_scatter((NCHUNK - 1) % 4)
    plsc.subcore_barrier()

    @pl.when(s < NT - 1)
    def _():
        pltpu.sync_copy(acc.at[pl.ds(r0, RPT_A)], out_hbm.at[pl.ds(off + r0, RPT_A)])

    @pl.when(s == NT - 1)
    def _():
        pltpu.sync_copy(acc.at[pl.ds(r0, RPT_B)], out_hbm.at[pl.ds(off + r0, RPT_B)])


def _scale_body(x_ref, degp_ref, xs_ref, normc_ref):
    d = degp_ref[0, :, 0:1] + degp_ref[1, :, 0:1]
    norm = lax.rsqrt(1.0 + d)
    normc_ref[...] = jnp.broadcast_to(norm, normc_ref.shape)
    xs = x_ref[...] * norm
    xs_ref[0] = xs[:, :FH]
    xs_ref[1] = xs[:, FH:]


def _mm_body(agg_ref, normc_ref, w1_ref, b1_ref, w2_ref, ps_ref):
    nc = normc_ref[...]
    a = jnp.concatenate([agg_ref[0] * nc, agg_ref[1] * nc], axis=1)
    h = jnp.dot(a, w1_ref[...], preferred_element_type=jnp.float32)
    h = jnp.maximum(h + b1_ref[...], 0.0)
    p = jnp.dot(h, w2_ref[...], preferred_element_type=jnp.float32)
    ps_ref[0] = p[:, :FH] * nc
    ps_ref[1] = p[:, FH:] * nc


def _final_body(agg_ref, normc_ref, b2_ref, out_ref):
    nc = normc_ref[...]
    out_ref[...] = jnp.concatenate(
        [agg_ref[0] * nc, agg_ref[1] * nc], axis=1) + b2_ref[...]


_BN = 1000
_G = N // _BN

_scale_call = pl.pallas_call(
    _scale_body,
    grid=(_G,),
    in_specs=[
        pl.BlockSpec((_BN, F_IN), lambda i: (i, 0)),
        pl.BlockSpec((2, _BN, FH), lambda i: (0, i, 0)),
    ],
    out_specs=[
        pl.BlockSpec((2, _BN, FH), lambda i: (0, i, 0)),
        pl.BlockSpec((_BN, FH), lambda i: (i, 0)),
    ],
    out_shape=[
        jax.ShapeDtypeStruct((2, N, FH), jnp.float32),
        jax.ShapeDtypeStruct((N, FH), jnp.float32),
    ],
)

_mm_call = pl.pallas_call(
    _mm_body,
    grid=(_G,),
    in_specs=[
        pl.BlockSpec((2, _BN, FH), lambda i: (0, i, 0)),
        pl.BlockSpec((_BN, FH), lambda i: (i, 0)),
        pl.BlockSpec((F_IN, HID), lambda i: (0, 0)),
        pl.BlockSpec((1, HID), lambda i: (0, 0)),
        pl.BlockSpec((HID, F_OUT), lambda i: (0, 0)),
    ],
    out_specs=pl.BlockSpec((2, _BN, FH), lambda i: (0, i, 0)),
    out_shape=jax.ShapeDtypeStruct((2, N, FH), jnp.float32),
)

_final_call = pl.pallas_call(
    _final_body,
    grid=(_G,),
    in_specs=[
        pl.BlockSpec((2, _BN, FH), lambda i: (0, i, 0)),
        pl.BlockSpec((_BN, FH), lambda i: (i, 0)),
        pl.BlockSpec((1, F_OUT), lambda i: (0, 0)),
    ],
    out_specs=pl.BlockSpec((_BN, F_OUT), lambda i: (i, 0)),
    out_shape=jax.ShapeDtypeStruct((N, F_OUT), jnp.float32),
)


def kernel(inputs, edge_index, W1, b1, W2, b2):
    src = edge_index[0].astype(jnp.int32)
    dst = edge_index[1].astype(jnp.int32)

    src_p = src.reshape(NT, NCHUNK, CH)
    src_p = jnp.pad(src_p, ((0, 0), (0, NCHB - NCHUNK), (0, 0)))
    dst_p = dst.reshape(NT, NCHUNK, CH)
    dst_p = jnp.pad(dst_p, ((0, 0), (0, NCHB - NCHUNK), (0, 0)))
    src_adj = jnp.stack([src_p, src_p + N])

    zerosw = jnp.zeros((N, FH), jnp.float32)
    onesw = jnp.ones((CH, FH), jnp.float32)

    degp = _deg_sc(dst_p, zerosw, onesw)
    xs2, normc = _scale_call(inputs, degp)
    agg1 = _agg_sc(xs2.reshape(2 * N, FH), src_adj, dst_p)
    ps2 = _mm_call(agg1.reshape(2, N, FH), normc, W1,
                   b1.reshape(1, HID), W2)
    agg2 = _agg_sc(ps2.reshape(2 * N, FH), src_adj, dst_p)
    out = _final_call(agg2.reshape(2, N, FH), normc,
                      b2.reshape(1, F_OUT))
    return out
